# baseline (speedup 1.0000x reference)
import functools

import jax
import jax.numpy as jnp
from jax import lax
from jax.experimental import pallas as pl
from jax.experimental.pallas import tpu as pltpu
from jax.experimental.pallas import tpu_sc as plsc

_ALPHA = 4


def kernel(frames):
    C, T, H, W = frames.shape
    TS = T // _ALPHA
    P = H * W
    src = frames.reshape(C * T, P)

    info = plsc.get_sparse_core_info()
    NC, NS = info.num_cores, info.num_subcores
    NW = NC * NS
    NSLICES = C * TS
    CHUNKS = 4
    NTASK = NSLICES * CHUNKS
    TPW = NTASK // NW
    CHUNK = P // CHUNKS

    mesh = plsc.VectorSubcoreMesh(core_axis_name="c", subcore_axis_name="s")

    @functools.partial(
        pl.kernel,
        mesh=mesh,
        out_type=jax.ShapeDtypeStruct((NSLICES, P), jnp.float32),
        scratch_types=[pltpu.SemaphoreType.DMA],
    )
    def gather_slices(src_hbm, out_hbm, sem):
        wid = lax.axis_index("s") * NC + lax.axis_index("c")
        copies = []
        for k in range(TPW):
            task = wid * TPW + k
            sl = task // CHUNKS
            q = task % CHUNKS
            c = sl // TS
            t = sl % TS
            src_row = c * T + (t * (T - 1)) // (TS - 1)
            off = q * CHUNK
            copies.append(pltpu.async_copy(
                src_hbm.at[src_row, pl.ds(off, CHUNK)],
                out_hbm.at[sl, pl.ds(off, CHUNK)],
                sem,
            ))
        for cp in copies:
            cp.wait()

    slow = gather_slices(src).reshape(C, TS, H, W)
    return (slow, frames)


# speedup vs baseline: 3.4279x; 3.4279x over previous
import functools

import jax
import jax.numpy as jnp
from jax import lax
from jax.experimental import pallas as pl
from jax.experimental.pallas import tpu as pltpu
from jax.experimental.pallas import tpu_sc as plsc

_ALPHA = 4


def kernel(frames):
    C, T, H, W = frames.shape
    TS = T // _ALPHA
    P = H * W
    src = frames.reshape(C * T, P)

    info = plsc.get_sparse_core_info()
    NC, NS = info.num_cores, info.num_subcores
    NW = NC * NS
    NSLICES = C * TS
    CHUNKS = 4
    NTASK = NSLICES * CHUNKS
    TPW = NTASK // NW
    CHUNK = P // CHUNKS

    mesh = plsc.VectorSubcoreMesh(core_axis_name="c", subcore_axis_name="s")

    @functools.partial(
        pl.kernel,
        mesh=mesh,
        out_type=jax.ShapeDtypeStruct((NSLICES, P), jnp.float32),
        scratch_types=[
            pltpu.VMEM((CHUNK,), jnp.float32),
            pltpu.VMEM((CHUNK,), jnp.float32),
            pltpu.SemaphoreType.DMA,
            pltpu.SemaphoreType.DMA,
            pltpu.SemaphoreType.DMA,
            pltpu.SemaphoreType.DMA,
        ],
    )
    def gather_slices(src_hbm, out_hbm, buf0, buf1, g0, g1, s0, s1):
        wid = lax.axis_index("s") * NC + lax.axis_index("c")
        bufs = (buf0, buf1)
        gsems = (g0, g1)
        ssems = (s0, s1)

        def task_refs(k):
            task = wid * TPW + k
            sl = task // CHUNKS
            q = task % CHUNKS
            c = sl // TS
            t = sl % TS
            src_row = c * T + (t * (T - 1)) // (TS - 1)
            off = q * CHUNK
            return (src_hbm.at[src_row, pl.ds(off, CHUNK)],
                    out_hbm.at[sl, pl.ds(off, CHUNK)])

        gathers = [None, None]
        scatters = [None, None]
        for k in range(TPW):
            b = k % 2
            src_ref, dst_ref = task_refs(k)
            if scatters[b] is not None:
                scatters[b].wait()
            gathers[b] = pltpu.async_copy(src_ref, bufs[b], gsems[b])
            gathers[b].wait()
            scatters[b] = pltpu.async_copy(bufs[b], dst_ref, ssems[b])
        for b in range(2):
            if scatters[b] is not None:
                scatters[b].wait()

    slow = gather_slices(src).reshape(C, TS, H, W)
    return (slow, frames)


# speedup vs baseline: 6.0032x; 1.7513x over previous
import functools

import jax
import jax.numpy as jnp
from jax import lax
from jax.experimental import pallas as pl
from jax.experimental.pallas import tpu as pltpu
from jax.experimental.pallas import tpu_sc as plsc

_ALPHA = 4


def kernel(frames):
    C, T, H, W = frames.shape
    TS = T // _ALPHA
    NSLICES = C * TS
    CHUNKS = 4
    RB = H // CHUNKS

    info = plsc.get_sparse_core_info()
    NC, NS = info.num_cores, info.num_subcores
    NW = NC * NS
    NTASK = NSLICES * CHUNKS
    TPW = NTASK // NW

    mesh = plsc.VectorSubcoreMesh(core_axis_name="c", subcore_axis_name="s")

    @functools.partial(
        pl.kernel,
        mesh=mesh,
        out_type=jax.ShapeDtypeStruct((C, TS, H, W), jnp.float32),
        scratch_types=[
            pltpu.VMEM((RB, W), jnp.float32),
            pltpu.VMEM((RB, W), jnp.float32),
            pltpu.SemaphoreType.DMA,
            pltpu.SemaphoreType.DMA,
            pltpu.SemaphoreType.DMA,
            pltpu.SemaphoreType.DMA,
        ],
        compiler_params=pltpu.CompilerParams(use_tc_tiling_on_sc=True),
    )
    def gather_slices(src_hbm, out_hbm, buf0, buf1, g0, g1, s0, s1):
        wid = lax.axis_index("s") * NC + lax.axis_index("c")
        bufs = (buf0, buf1)
        gsems = (g0, g1)
        ssems = (s0, s1)

        def task_refs(k):
            task = wid * TPW + k
            sl = task // CHUNKS
            q = task % CHUNKS
            c = sl // TS
            t = sl % TS
            t_src = (t * (T - 1)) // (TS - 1)
            rows = pl.ds(q * RB, RB)
            return (src_hbm.at[c, t_src, rows, :],
                    out_hbm.at[c, t, rows, :])

        gathers = [None, None]
        scatters = [None, None]
        for k in range(TPW):
            b = k % 2
            src_ref, dst_ref = task_refs(k)
            if scatters[b] is not None:
                scatters[b].wait()
            gathers[b] = pltpu.async_copy(src_ref, bufs[b], gsems[b])
            gathers[b].wait()
            scatters[b] = pltpu.async_copy(bufs[b], dst_ref, ssems[b])
        for b in range(2):
            if scatters[b] is not None:
                scatters[b].wait()

    slow = gather_slices(frames)
    return (slow, frames)
